# baseline (speedup 1.0000x reference)
import functools

import jax
import jax.numpy as jnp
from jax.experimental import pallas as pl

B = 4096
D = 768
M = 16
R = 48
MR = M * R


def _moe_block(x_ref, w_enc_ref, b_enc_ref, w_gate_ref, uf_ref, vf_ref,
               gamma_ref, out_ref):
    x = x_ref[...]
    h = jax.lax.dot_general(x, w_enc_ref[...], (((1,), (1,)), ((), ())),
                            preferred_element_type=jnp.float32)
    h = h + b_enc_ref[...]

    logits = jax.lax.dot_general(h, w_gate_ref[...], (((1,), (1,)), ((), ())),
                                 preferred_element_type=jnp.float32)
    bt = logits.shape[0]
    lane = jax.lax.broadcasted_iota(jnp.int32, (bt, M), 1)
    v1 = jnp.max(logits, axis=1, keepdims=True)
    idx1 = jnp.min(jnp.where(logits == v1, lane, M), axis=1, keepdims=True)
    hot1 = lane == idx1
    masked = jnp.where(hot1, -jnp.inf, logits)
    v2 = jnp.max(masked, axis=1, keepdims=True)
    idx2 = jnp.min(jnp.where(masked == v2, lane, M), axis=1, keepdims=True)
    e2 = jnp.exp(v2 - v1)
    denom = 1.0 + e2 + 1e-12
    w1 = 1.0 / denom
    w2 = e2 / denom

    s = jax.lax.dot_general(h, uf_ref[...], (((1,), (1,)), ((), ())),
                            preferred_element_type=jnp.float32)
    a = s * jax.lax.logistic(s)

    elane = jax.lax.broadcasted_iota(jnp.int32, (bt, MR), 1) // R
    scale = (jnp.where(elane == idx1, w1, 0.0)
             + jnp.where(elane == idx2, w2, 0.0)) * gamma_ref[...]
    w = a * scale

    y = jax.lax.dot_general(w, vf_ref[...], (((1,), (0,)), ((), ())),
                            preferred_element_type=jnp.float32)
    out_ref[...] = h + y


@functools.partial(jax.jit, static_argnames=("bt", "interpret"))
def _moe(x, w_enc, b_enc2, w_gate, uf, vf, gammaf, bt=2048, interpret=False):
    grid = x.shape[0] // bt
    return pl.pallas_call(
        _moe_block,
        grid=(grid,),
        in_specs=[
            pl.BlockSpec((bt, D), lambda i: (i, 0)),
            pl.BlockSpec((D, D), lambda i: (0, 0)),
            pl.BlockSpec((1, D), lambda i: (0, 0)),
            pl.BlockSpec((M, D), lambda i: (0, 0)),
            pl.BlockSpec((MR, D), lambda i: (0, 0)),
            pl.BlockSpec((MR, D), lambda i: (0, 0)),
            pl.BlockSpec((1, MR), lambda i: (0, 0)),
        ],
        out_specs=pl.BlockSpec((bt, D), lambda i: (i, 0)),
        out_shape=jax.ShapeDtypeStruct((x.shape[0], D), jnp.float32),
        interpret=interpret,
    )(x, w_enc, b_enc2, w_gate, uf, vf, gammaf)


def kernel(x, W_enc, b_enc, W_gate, U, V, gamma):
    uf = U.reshape(MR, D)
    vf = V.reshape(MR, D)
    gammaf = jnp.repeat(gamma, R).reshape(1, MR)
    return _moe(x, W_enc, b_enc.reshape(1, D), W_gate, uf, vf, gammaf)


# speedup vs baseline: 1.0935x; 1.0935x over previous
import functools

import jax
import jax.numpy as jnp
from jax.experimental import pallas as pl

B = 4096
D = 768
M = 16
R = 48
MR = M * R


def _moe_block(x_ref, w_enc_ref, b_enc_ref, w_gate_ref, uf_ref, vf_ref,
               gamma_ref, out_ref):
    x = x_ref[...]
    w_enc = w_enc_ref[...]
    w_gate_f = jax.lax.dot_general(w_gate_ref[...], w_enc,
                                   (((1,), (0,)), ((), ())),
                                   preferred_element_type=jnp.float32)
    b_gate = jax.lax.dot_general(b_enc_ref[...], w_gate_ref[...],
                                 (((1,), (1,)), ((), ())),
                                 preferred_element_type=jnp.float32)
    logits = jax.lax.dot_general(x, w_gate_f, (((1,), (1,)), ((), ())),
                                 preferred_element_type=jnp.float32) + b_gate

    h = jax.lax.dot_general(x, w_enc, (((1,), (1,)), ((), ())),
                            preferred_element_type=jnp.float32)
    h = h + b_enc_ref[...]
    bt = logits.shape[0]
    lane = jax.lax.broadcasted_iota(jnp.int32, (bt, M), 1)
    v1 = jnp.max(logits, axis=1, keepdims=True)
    idx1 = jnp.min(jnp.where(logits == v1, lane, M), axis=1, keepdims=True)
    hot1 = lane == idx1
    masked = jnp.where(hot1, -jnp.inf, logits)
    v2 = jnp.max(masked, axis=1, keepdims=True)
    idx2 = jnp.min(jnp.where(masked == v2, lane, M), axis=1, keepdims=True)
    e2 = jnp.exp(v2 - v1)
    denom = 1.0 + e2 + 1e-12
    w1 = 1.0 / denom
    w2 = e2 / denom

    s = jax.lax.dot_general(h, uf_ref[...], (((1,), (1,)), ((), ())),
                            preferred_element_type=jnp.float32)
    a = s * jax.lax.logistic(s)

    elane = jax.lax.broadcasted_iota(jnp.int32, (bt, MR), 1) // R
    scale = (jnp.where(elane == idx1, w1, 0.0)
             + jnp.where(elane == idx2, w2, 0.0)) * gamma_ref[...]
    w = a * scale

    y = jax.lax.dot_general(w, vf_ref[...], (((1,), (0,)), ((), ())),
                            preferred_element_type=jnp.float32)
    out_ref[...] = h + y


@functools.partial(jax.jit, static_argnames=("bt", "interpret"))
def _moe(x, w_enc, b_enc2, w_gate, uf, vf, gammaf, bt=1024, interpret=False):
    grid = x.shape[0] // bt
    return pl.pallas_call(
        _moe_block,
        grid=(grid,),
        in_specs=[
            pl.BlockSpec((bt, D), lambda i: (i, 0)),
            pl.BlockSpec((D, D), lambda i: (0, 0)),
            pl.BlockSpec((1, D), lambda i: (0, 0)),
            pl.BlockSpec((M, D), lambda i: (0, 0)),
            pl.BlockSpec((MR, D), lambda i: (0, 0)),
            pl.BlockSpec((MR, D), lambda i: (0, 0)),
            pl.BlockSpec((1, MR), lambda i: (0, 0)),
        ],
        out_specs=pl.BlockSpec((bt, D), lambda i: (i, 0)),
        out_shape=jax.ShapeDtypeStruct((x.shape[0], D), jnp.float32),
        interpret=interpret,
    )(x, w_enc, b_enc2, w_gate, uf, vf, gammaf)


def kernel(x, W_enc, b_enc, W_gate, U, V, gamma):
    uf = U.reshape(MR, D)
    vf = V.reshape(MR, D)
    gammaf = jnp.repeat(gamma, R).reshape(1, MR)
    return _moe(x, W_enc, b_enc.reshape(1, D), W_gate, uf, vf, gammaf)
